# baseline (speedup 1.0000x reference)
import jax
import jax.numpy as jnp
from jax import lax
from jax.experimental import pallas as pl
from jax.experimental.pallas import tpu as pltpu

B, T, S = 64, 4096, 16
BB = 8
BIG = 65536.0
N_ELEMS = float(B * T * S)


def _bce_kernel(pred_ref, tgt_ref, out_ref, acc_ref):
    g = pl.program_id(0)

    @pl.when(g == 0)
    def _():
        acc_ref[0, 0] = 0.0

    tval = lax.broadcasted_iota(jnp.int32, (S, T), 1).astype(jnp.float32)
    jcol = lax.broadcasted_iota(jnp.int32, (S, S), 1).astype(jnp.float32)
    i_col = lax.broadcasted_iota(jnp.int32, (S, 1), 0).astype(jnp.float32)
    eye = (lax.broadcasted_iota(jnp.int32, (S, S), 0) ==
           lax.broadcasted_iota(jnp.int32, (S, S), 1)).astype(jnp.float32)

    total = jnp.zeros((), jnp.float32)
    for bb in range(BB):
        p = pred_ref[bb]
        t = tgt_ref[bb]

        lp = jnp.maximum(jnp.log(p), -100.0)
        l1p = jnp.maximum(jnp.log(1.0 - p), -100.0)
        d = lp - l1p

        m16 = lax.dot_general(t, d, (((1,), (1,)), ((), ())),
                              preferred_element_type=jnp.float32)

        cand = jnp.where(t > 0.0, tval, BIG)
        o_col = jnp.min(cand, axis=1, keepdims=True)

        k_col = o_col * 16.0 + i_col
        kcol = jnp.broadcast_to(k_col, (S, S))
        krow = lax.dot_general(kcol, eye, (((0,), (0,)), ((), ())),
                               preferred_element_type=jnp.float32)
        less = (krow < kcol).astype(jnp.float32)
        rank = jnp.sum(less, axis=1, keepdims=True)
        perm = (rank == jcol).astype(jnp.float32)

        total = total - jnp.sum(l1p) - jnp.sum(m16 * perm)

    acc_ref[0, 0] = acc_ref[0, 0] + total

    @pl.when(g == B // BB - 1)
    def _():
        out_ref[...] = jnp.reshape(acc_ref[0, 0] * (1.0 / N_ELEMS), (1, 1))


@jax.jit
def kernel(predictions, targets):
    pr = jnp.transpose(predictions, (0, 2, 1))
    tg = jnp.transpose(targets, (0, 2, 1))
    spec = pl.BlockSpec((BB, S, T), lambda b: (b, 0, 0))
    out = pl.pallas_call(
        _bce_kernel,
        grid=(B // BB,),
        in_specs=[spec, spec],
        out_specs=pl.BlockSpec((1, 1), lambda b: (0, 0)),
        out_shape=jax.ShapeDtypeStruct((1, 1), jnp.float32),
        scratch_shapes=[pltpu.SMEM((1, 1), jnp.float32)],
    )(pr, tg)
    return out[0, 0]


# speedup vs baseline: 1.1297x; 1.1297x over previous
import jax
import jax.numpy as jnp
from jax import lax
from jax.experimental import pallas as pl
from jax.experimental.pallas import tpu as pltpu

B, T, S = 64, 4096, 16
BB = 16
BIG = 65536.0
N_ELEMS = float(B * T * S)


def _bce_kernel(pred_ref, tgt_ref, out_ref, acc_ref):
    g = pl.program_id(0)

    @pl.when(g == 0)
    def _():
        acc_ref[0, 0] = 0.0

    tval = lax.broadcasted_iota(jnp.int32, (S, T), 1).astype(jnp.float32)
    jcol = lax.broadcasted_iota(jnp.int32, (S, S), 1).astype(jnp.float32)
    i_col = lax.broadcasted_iota(jnp.int32, (S, 1), 0).astype(jnp.float32)
    eye = (lax.broadcasted_iota(jnp.int32, (S, S), 0) ==
           lax.broadcasted_iota(jnp.int32, (S, S), 1)).astype(jnp.float32)

    total = jnp.zeros((), jnp.float32)
    for bb in range(BB):
        p = pred_ref[bb]
        t = tgt_ref[bb]

        lp = jnp.maximum(jnp.log(p), -100.0)
        l1p = jnp.maximum(jnp.log(1.0 - p), -100.0)
        d = lp - l1p

        m16 = lax.dot_general(t, d, (((1,), (1,)), ((), ())),
                              preferred_element_type=jnp.float32)

        cand = jnp.where(t > 0.0, tval, BIG)
        o_col = jnp.min(cand, axis=1, keepdims=True)

        k_col = o_col * 16.0 + i_col
        kcol = jnp.broadcast_to(k_col, (S, S))
        krow = lax.dot_general(kcol, eye, (((0,), (0,)), ((), ())),
                               preferred_element_type=jnp.float32)
        less = (krow < kcol).astype(jnp.float32)
        rank = jnp.sum(less, axis=1, keepdims=True)
        perm = (rank == jcol).astype(jnp.float32)

        total = total - jnp.sum(l1p) - jnp.sum(m16 * perm)

    acc_ref[0, 0] = acc_ref[0, 0] + total

    @pl.when(g == B // BB - 1)
    def _():
        out_ref[...] = jnp.reshape(acc_ref[0, 0] * (1.0 / N_ELEMS), (1, 1))


@jax.jit
def kernel(predictions, targets):
    pr = jnp.transpose(predictions, (0, 2, 1))
    tg = jnp.transpose(targets, (0, 2, 1))
    spec = pl.BlockSpec((BB, S, T), lambda b: (b, 0, 0))
    out = pl.pallas_call(
        _bce_kernel,
        grid=(B // BB,),
        in_specs=[spec, spec],
        out_specs=pl.BlockSpec((1, 1), lambda b: (0, 0)),
        out_shape=jax.ShapeDtypeStruct((1, 1), jnp.float32),
        scratch_shapes=[pltpu.SMEM((1, 1), jnp.float32)],
    )(pr, tg)
    return out[0, 0]
